# baseline (speedup 1.0000x reference)
import functools

import jax
import jax.numpy as jnp
from jax import lax
from jax.experimental import pallas as pl
from jax.experimental.pallas import tpu as pltpu
from jax.experimental.pallas import tpu_sc as plsc

_CHUNK = 512
_NBUF = 2


@functools.lru_cache(maxsize=None)
def _make_gather(N, S, D):
    info = plsc.get_sparse_core_info()
    nc, ns = info.num_cores, info.num_subcores
    nw = nc * ns
    chunks_per_s = N // _CHUNK
    n_chunks = S * chunks_per_s
    c_per_w = n_chunks // nw
    assert c_per_w % _NBUF == 0
    mesh = plsc.VectorSubcoreMesh(core_axis_name="c", subcore_axis_name="s")

    scratch = []
    for _ in range(_NBUF):
        scratch += [
            pltpu.VMEM((_CHUNK,), jnp.int32),
            pltpu.VMEM((_CHUNK, D), jnp.float32),
            pltpu.VMEM((D, _CHUNK), jnp.float32),
            pltpu.SemaphoreType.DMA,
            pltpu.SemaphoreType.DMA,
        ]

    @functools.partial(
        pl.kernel,
        mesh=mesh,
        out_type=jax.ShapeDtypeStruct((S, D, N), jnp.float32),
        scratch_types=scratch,
        compiler_params=pltpu.CompilerParams(
            use_tc_tiling_on_sc=False,
            needs_layout_passes=False,
            disable_bounds_checks=True,
        ),
    )
    def gather_kernel(idx_hbm, table_hbm, out_hbm, *bufs):
        rings = [tuple(bufs[5 * b : 5 * b + 5]) for b in range(_NBUF)]
        iota = lax.iota(jnp.int32, 16)
        wid = lax.axis_index("s") * nc + lax.axis_index("c")
        c_base = wid * c_per_w

        def stage_in(c, b):
            idx_v, rows_v, _, gsem, _ = rings[b]
            pltpu.sync_copy(idx_hbm.at[pl.ds(c * _CHUNK, _CHUNK)], idx_v)
            pltpu.make_async_copy(table_hbm.at[idx_v], rows_v, gsem).start()

        def out_view(c):
            s = c // chunks_per_s
            b0 = (c % chunks_per_s) * _CHUNK
            return out_hbm.at[s, :, pl.ds(b0, _CHUNK)]

        for b in range(_NBUF):
            stage_in(c_base + b, b)

        def body(ii, carry):
            for bb in range(_NBUF):
                i = ii * _NBUF + bb
                c = c_base + i
                idx_v, rows_v, td_v, gsem, ssem = rings[bb]
                pltpu.make_async_copy(table_hbm.at[idx_v], rows_v, gsem).wait()

                @pl.when(i >= _NBUF)
                def _():
                    pltpu.make_async_copy(td_v, out_view(c - _NBUF), ssem).wait()

                @pl.loop(0, _CHUNK // 16, unroll=4)
                def _grp(g):
                    r16 = g * 16 + iota
                    for d in range(D):
                        vec = plsc.load_gather(
                            rows_v, [r16, jnp.full((16,), d, jnp.int32)]
                        )
                        td_v[d, pl.ds(g * 16, 16)] = vec

                pltpu.make_async_copy(td_v, out_view(c), ssem).start()

                @pl.when(i + _NBUF < c_per_w)
                def _():
                    stage_in(c + _NBUF, bb)

            return carry

        lax.fori_loop(0, c_per_w // _NBUF, body, 0)

        for b in range(_NBUF):
            _, _, td_v, _, ssem = rings[b]
            pltpu.make_async_copy(
                td_v, out_hbm.at[0, :, pl.ds(0, _CHUNK)], ssem
            ).wait()

    return gather_kernel


def kernel(indices, weight):
    n, s = indices.shape
    v, d = weight.shape
    flat_idx = indices.T.reshape(s * n)
    out = _make_gather(n, s, d)(flat_idx, weight)
    return out.transpose(2, 0, 1)


# speedup vs baseline: 1.8239x; 1.8239x over previous
import functools

import jax
import jax.numpy as jnp
from jax import lax
from jax.experimental import pallas as pl
from jax.experimental.pallas import tpu as pltpu
from jax.experimental.pallas import tpu_sc as plsc

_CHUNK = 512
_NBUF = 2


@functools.lru_cache(maxsize=None)
def _make_gather(N, S, D):
    info = plsc.get_sparse_core_info()
    nc, ns = info.num_cores, info.num_subcores
    nw = nc * ns
    chunks_per_s = N // _CHUNK
    n_chunks = S * chunks_per_s
    c_per_w = n_chunks // nw
    assert c_per_w % _NBUF == 0
    mesh = plsc.VectorSubcoreMesh(core_axis_name="c", subcore_axis_name="s")

    scratch = []
    for _ in range(_NBUF):
        scratch += [
            pltpu.VMEM((_CHUNK,), jnp.int32),
            pltpu.VMEM((_CHUNK, D), jnp.float32),
            pltpu.SemaphoreType.DMA,
            pltpu.SemaphoreType.DMA,
        ]

    @functools.partial(
        pl.kernel,
        mesh=mesh,
        out_type=jax.ShapeDtypeStruct((S, N, D), jnp.float32),
        scratch_types=scratch,
        compiler_params=pltpu.CompilerParams(use_tc_tiling_on_sc=False),
    )
    def gather_kernel(idx_hbm, table_hbm, out_hbm, *bufs):
        rings = [tuple(bufs[4 * b : 4 * b + 4]) for b in range(_NBUF)]
        wid = lax.axis_index("s") * nc + lax.axis_index("c")
        c_base = wid * c_per_w

        def stage_in(c, b):
            idx_v, rows_v, gsem, _ = rings[b]
            pltpu.sync_copy(idx_hbm.at[pl.ds(c * _CHUNK, _CHUNK)], idx_v)
            pltpu.make_async_copy(table_hbm.at[idx_v], rows_v, gsem).start()

        def out_view(c):
            s = c // chunks_per_s
            b0 = (c % chunks_per_s) * _CHUNK
            return out_hbm.at[s, pl.ds(b0, _CHUNK), :]

        for b in range(_NBUF):
            stage_in(c_base + b, b)

        def body(ii, carry):
            for bb in range(_NBUF):
                i = ii * _NBUF + bb
                c = c_base + i
                idx_v, rows_v, gsem, ssem = rings[bb]
                pltpu.make_async_copy(table_hbm.at[idx_v], rows_v, gsem).wait()
                pltpu.make_async_copy(rows_v, out_view(c), ssem).start()

                @pl.when(i + _NBUF < c_per_w)
                def _():
                    pltpu.sync_copy(
                        idx_hbm.at[pl.ds((c + _NBUF) * _CHUNK, _CHUNK)], idx_v
                    )
                    pltpu.make_async_copy(rows_v, out_view(c), ssem).wait()
                    pltpu.make_async_copy(
                        table_hbm.at[idx_v], rows_v, gsem
                    ).start()

            return carry

        lax.fori_loop(0, c_per_w // _NBUF, body, 0)

        for b in range(_NBUF):
            _, rows_v, _, ssem = rings[b]
            pltpu.make_async_copy(
                rows_v, out_hbm.at[0, pl.ds(0, _CHUNK), :], ssem
            ).wait()

    return gather_kernel


@functools.lru_cache(maxsize=None)
def _make_otranspose(N, S, D):
    rows_in = N * D // 128

    nq = N // 4

    def body(x_ref, out_ref):
        xt = x_ref[...].T
        for k in range(4):
            out_ref[0, :, k * nq:(k + 1) * nq] = xt[32 * k:32 * k + 32, :]

    return pl.pallas_call(
        body,
        grid=(S,),
        in_specs=[pl.BlockSpec((rows_in, 128), lambda s: (s, 0))],
        out_specs=pl.BlockSpec((1, D, N), lambda s: (s, 0, 0)),
        out_shape=jax.ShapeDtypeStruct((S, D, N), jnp.float32),
    )


def kernel(indices, weight):
    n, s = indices.shape
    v, d = weight.shape
    flat_idx = indices.T.reshape(s, 4, n // 4).transpose(0, 2, 1).reshape(s * n)
    out = _make_gather(n, s, d)(flat_idx, weight)
    o_t = _make_otranspose(n, s, d)(out.reshape(s * n * d // 128, 128))
    return o_t.transpose(2, 0, 1)


# speedup vs baseline: 3.0617x; 1.6787x over previous
import functools

import jax
import jax.numpy as jnp
from jax import lax
from jax.experimental import pallas as pl
from jax.experimental.pallas import tpu as pltpu
from jax.experimental.pallas import tpu_sc as plsc

_CHUNK = 512
_NBUF = 2


@functools.lru_cache(maxsize=None)
def _make_gather(N, S, D):
    info = plsc.get_sparse_core_info()
    nc, ns = info.num_cores, info.num_subcores
    nw = nc * ns
    chunks_per_s = N // _CHUNK
    n_chunks = S * chunks_per_s
    c_per_w = n_chunks // nw
    assert c_per_w % _NBUF == 0
    mesh = plsc.VectorSubcoreMesh(core_axis_name="c", subcore_axis_name="s")

    scratch = []
    for _ in range(_NBUF):
        scratch += [
            pltpu.VMEM((_CHUNK,), jnp.int32),
            pltpu.VMEM((_CHUNK, D), jnp.float32),
            pltpu.SemaphoreType.DMA,
            pltpu.SemaphoreType.DMA,
        ]

    @functools.partial(
        pl.kernel,
        mesh=mesh,
        out_type=jax.ShapeDtypeStruct((S, N, D), jnp.float32),
        scratch_types=scratch,
        compiler_params=pltpu.CompilerParams(use_tc_tiling_on_sc=False),
    )
    def gather_kernel(idx_hbm, table_hbm, out_hbm, *bufs):
        rings = [tuple(bufs[4 * b : 4 * b + 4]) for b in range(_NBUF)]
        wid = lax.axis_index("s") * nc + lax.axis_index("c")
        c_base = wid * c_per_w

        def stage_in(c, b):
            idx_v, rows_v, gsem, _ = rings[b]
            pltpu.sync_copy(idx_hbm.at[pl.ds(c * _CHUNK, _CHUNK)], idx_v)
            pltpu.make_async_copy(table_hbm.at[idx_v], rows_v, gsem).start()

        def out_view(c):
            s = c // chunks_per_s
            b0 = (c % chunks_per_s) * _CHUNK
            return out_hbm.at[s, pl.ds(b0, _CHUNK), :]

        for b in range(_NBUF):
            stage_in(c_base + b, b)

        def body(ii, carry):
            for bb in range(_NBUF):
                i = ii * _NBUF + bb
                c = c_base + i
                idx_v, rows_v, gsem, ssem = rings[bb]
                pltpu.make_async_copy(table_hbm.at[idx_v], rows_v, gsem).wait()
                pltpu.make_async_copy(rows_v, out_view(c), ssem).start()

                @pl.when(i + _NBUF < c_per_w)
                def _():
                    pltpu.sync_copy(
                        idx_hbm.at[pl.ds((c + _NBUF) * _CHUNK, _CHUNK)], idx_v
                    )
                    pltpu.make_async_copy(rows_v, out_view(c), ssem).wait()
                    pltpu.make_async_copy(
                        table_hbm.at[idx_v], rows_v, gsem
                    ).start()

            return carry

        lax.fori_loop(0, c_per_w // _NBUF, body, 0)

        for b in range(_NBUF):
            _, rows_v, _, ssem = rings[b]
            pltpu.make_async_copy(
                rows_v, out_hbm.at[0, pl.ds(0, _CHUNK), :], ssem
            ).wait()

    return gather_kernel


@functools.lru_cache(maxsize=None)
def _make_wtranspose(V, D):
    cols = 16384
    grid = (V + cols - 1) // cols
    nq = cols // 4

    def body(x_ref, out_ref):
        x = x_ref[...]
        y = jnp.concatenate(
            [x[:, k * nq:(k + 1) * nq] for k in range(4)], axis=0
        )
        out_ref[...] = y.T

    return pl.pallas_call(
        body,
        grid=(grid,),
        in_specs=[pl.BlockSpec((D, cols), lambda c: (0, c))],
        out_specs=pl.BlockSpec((nq, 128), lambda c: (c, 0)),
        out_shape=jax.ShapeDtypeStruct((grid * nq, 128), jnp.float32),
    )


@functools.lru_cache(maxsize=None)
def _make_otranspose(N, S, D):
    rows_in = N * D // 128

    nq = N // 4

    def body(x_ref, out_ref):
        xt = x_ref[...].T
        for k in range(4):
            out_ref[0, :, k * nq:(k + 1) * nq] = xt[32 * k:32 * k + 32, :]

    return pl.pallas_call(
        body,
        grid=(S,),
        in_specs=[pl.BlockSpec((rows_in, 128), lambda s: (s, 0))],
        out_specs=pl.BlockSpec((1, D, N), lambda s: (s, 0, 0)),
        out_shape=jax.ShapeDtypeStruct((S, D, N), jnp.float32),
    )


def kernel(indices, weight):
    n, s = indices.shape
    v, d = weight.shape
    flat_idx = indices.T.reshape(s, 4, n // 4).transpose(0, 2, 1).reshape(s * n)
    flat_idx = (
        (flat_idx >> 14 << 14) | ((flat_idx & 4095) << 2) | ((flat_idx >> 12) & 3)
    )
    w_r = _make_wtranspose(v, d)(weight.T)
    table = w_r.reshape(w_r.shape[0] * 128 // d, d)
    out = _make_gather(n, s, d)(flat_idx, table)
    o_t = _make_otranspose(n, s, d)(out.reshape(s * n * d // 128, 128))
    return o_t.transpose(2, 0, 1)


# speedup vs baseline: 4.5618x; 1.4899x over previous
import functools

import jax
import jax.numpy as jnp
from jax import lax
from jax.experimental import pallas as pl
from jax.experimental.pallas import tpu as pltpu
from jax.experimental.pallas import tpu_sc as plsc

_CHUNK = 512
_NBUF = 2


@functools.lru_cache(maxsize=None)
def _make_gather(N, S, D):
    info = plsc.get_sparse_core_info()
    nc, ns = info.num_cores, info.num_subcores
    nw = nc * ns
    chunks_per_s = N // _CHUNK
    n_chunks = S * chunks_per_s
    c_per_w = n_chunks // nw
    assert c_per_w % _NBUF == 0
    mesh = plsc.VectorSubcoreMesh(core_axis_name="c", subcore_axis_name="s")

    scratch = []
    for _ in range(_NBUF):
        scratch += [
            pltpu.VMEM((_CHUNK,), jnp.int32),
            pltpu.VMEM((_CHUNK, D), jnp.float32),
            pltpu.SemaphoreType.DMA,
            pltpu.SemaphoreType.DMA,
        ]

    @functools.partial(
        pl.kernel,
        mesh=mesh,
        out_type=jax.ShapeDtypeStruct((S, N // 4, 4, D), jnp.float32),
        scratch_types=scratch,
        compiler_params=pltpu.CompilerParams(use_tc_tiling_on_sc=False),
    )
    def gather_kernel(idx_hbm, table_hbm, out_hbm, *bufs):
        rings = [tuple(bufs[4 * b : 4 * b + 4]) for b in range(_NBUF)]
        wid = lax.axis_index("s") * nc + lax.axis_index("c")
        c_base = wid * c_per_w

        def stage_in(c, b):
            idx_v, rows_v, gsem, _ = rings[b]
            pltpu.sync_copy(idx_hbm.at[pl.ds(c * _CHUNK, _CHUNK)], idx_v)
            pltpu.make_async_copy(table_hbm.at[idx_v], rows_v, gsem).start()

        def out_view(c):
            s = c // chunks_per_s
            local = c % chunks_per_s
            k0 = local // (chunks_per_s // 4)
            q0 = (local % (chunks_per_s // 4)) * _CHUNK
            return out_hbm.at[s, pl.ds(q0, _CHUNK), k0, :]

        for b in range(_NBUF):
            stage_in(c_base + b, b)

        def body(ii, carry):
            for bb in range(_NBUF):
                i = ii * _NBUF + bb
                c = c_base + i
                idx_v, rows_v, gsem, ssem = rings[bb]
                pltpu.make_async_copy(table_hbm.at[idx_v], rows_v, gsem).wait()
                pltpu.make_async_copy(rows_v, out_view(c), ssem).start()

                @pl.when(i + _NBUF < c_per_w)
                def _():
                    pltpu.sync_copy(
                        idx_hbm.at[pl.ds((c + _NBUF) * _CHUNK, _CHUNK)], idx_v
                    )
                    pltpu.make_async_copy(rows_v, out_view(c), ssem).wait()
                    pltpu.make_async_copy(
                        table_hbm.at[idx_v], rows_v, gsem
                    ).start()

            return carry

        lax.fori_loop(0, c_per_w // _NBUF, body, 0)

        for b in range(_NBUF):
            _, rows_v, _, ssem = rings[b]
            pltpu.make_async_copy(
                rows_v, out_hbm.at[0, pl.ds(0, _CHUNK), 0, :], ssem
            ).wait()

    return gather_kernel


@functools.lru_cache(maxsize=None)
def _make_wtranspose(V, D):
    cols = 16384
    grid = (V + cols - 1) // cols
    nq = cols // 4

    def body(x_ref, out_ref):
        x = x_ref[...]
        y = jnp.concatenate(
            [x[:, k * nq:(k + 1) * nq] for k in range(4)], axis=0
        )
        out_ref[...] = y.T

    return pl.pallas_call(
        body,
        grid=(grid,),
        in_specs=[pl.BlockSpec((D, cols), lambda c: (0, c))],
        out_specs=pl.BlockSpec((nq, 128), lambda c: (c, 0)),
        out_shape=jax.ShapeDtypeStruct((grid * nq, 128), jnp.float32),
    )


@functools.lru_cache(maxsize=None)
def _make_otranspose(N, S, D):
    rows_in = N * D // 128

    nq = N // 4

    def body(x_ref, out_ref):
        xt = x_ref[...].T
        for k in range(4):
            out_ref[0, :, k * nq:(k + 1) * nq] = xt[32 * k:32 * k + 32, :]

    return pl.pallas_call(
        body,
        grid=(S,),
        in_specs=[pl.BlockSpec((rows_in, 128), lambda s: (s, 0))],
        out_specs=pl.BlockSpec((1, D, N), lambda s: (s, 0, 0)),
        out_shape=jax.ShapeDtypeStruct((S, D, N), jnp.float32),
    )


def kernel(indices, weight):
    n, s = indices.shape
    v, d = weight.shape
    flat_idx = indices.T.reshape(s * n)
    flat_idx = (
        (flat_idx >> 14 << 14) | ((flat_idx & 4095) << 2) | ((flat_idx >> 12) & 3)
    )
    w_r = _make_wtranspose(v, d)(weight.T)
    table = w_r.reshape(w_r.shape[0] * 128 // d, d)
    out = _make_gather(n, s, d)(flat_idx, table)
    o_t = _make_otranspose(n, s, d)(out.reshape(s * n * d // 128, 128))
    return o_t.transpose(2, 0, 1)
